# baseline (speedup 1.0000x reference)
import functools
from itertools import combinations

import jax
import jax.numpy as jnp
from jax import lax
from jax.experimental import pallas as pl
from jax.experimental.pallas import tpu as pltpu
from jax.experimental.pallas import tpu_sc as plsc

B = 4096
F = 26
K = 16
V = 1000000
NW = 32
NPER = (B * F) // NW
_PAIRS = list(combinations(range(F), 2))
NPAIR = len(_PAIRS)
BN_EPS = 1e-3


def _sc_gather_body(idx_hbm, xv_hbm, xw_hbm, xv_out, xw_out,
                    idx_v, rows_v, w_v, sem1, sem2):
    wid = lax.axis_index("s") * 2 + lax.axis_index("c")
    base = wid * NPER
    pltpu.sync_copy(idx_hbm.at[pl.ds(base, NPER)], idx_v)
    cp1 = pltpu.async_copy(xv_hbm.at[idx_v], rows_v, sem1)
    cp2 = pltpu.async_copy(xw_hbm.at[idx_v], w_v, sem2)
    cp1.wait()
    cp2.wait()
    pltpu.sync_copy(rows_v, xv_out.at[pl.ds(base, NPER)])
    pltpu.sync_copy(w_v, xw_out.at[pl.ds(base, NPER)])


def _sc_gather(idx_flat, xv_table, xw2):
    mesh = plsc.VectorSubcoreMesh(core_axis_name="c", subcore_axis_name="s")
    f = pl.kernel(
        _sc_gather_body,
        out_type=[
            jax.ShapeDtypeStruct((B * F, K), jnp.float32),
            jax.ShapeDtypeStruct((B * F,), jnp.float32),
        ],
        mesh=mesh,
        compiler_params=pltpu.CompilerParams(use_tc_tiling_on_sc=False),
        scratch_types=[
            pltpu.VMEM((NPER,), jnp.int32),
            pltpu.VMEM((NPER, K), jnp.float32),
            pltpu.VMEM((NPER,), jnp.float32),
            pltpu.SemaphoreType.DMA,
            pltpu.SemaphoreType.DMA,
        ],
    )
    return f(idx_flat, xv_table, xw2)


def _tc_body(xv_ref, xw_ref, w1_ref, b1_ref, w2_ref, b2_ref, w3_ref,
             b3_ref, gam_ref, bet_ref, ew_ref, out_ref, xt_ref):
    for f in range(F):
        xt_ref[f * K:(f + 1) * K, :] = jnp.transpose(
            xv_ref[:, f * K:(f + 1) * K])

    fm = jnp.zeros((B,), dtype=jnp.float32)
    const = jnp.float32(0.0)
    inv_b = jnp.float32(1.0 / B)
    for p, (r, c) in enumerate(_PAIRS):
        a = xt_ref[r * K:(r + 1) * K, :]
        b = xt_ref[c * K:(c + 1) * K, :]
        row = jnp.sum(a * b, axis=0)
        s1 = jnp.sum(row) * inv_b
        var = jnp.sum((row - s1) ** 2) * inv_b
        rstd = lax.rsqrt(var + BN_EPS)
        cw = gam_ref[p] * ew_ref[p] * rstd
        fm = fm + cw * row
        const = const + ew_ref[p] * bet_ref[p] - cw * s1

    lin = jnp.sum(xw_ref[...], axis=1)

    xt = xt_ref[...]
    dn = (((0,), (0,)), ((), ()))
    h1 = lax.dot_general(w1_ref[...], xt, dn,
                         preferred_element_type=jnp.float32)
    h1 = jnp.maximum(h1 + b1_ref[...], 0.0)
    h2 = lax.dot_general(w2_ref[...], h1, dn,
                         preferred_element_type=jnp.float32)
    h2 = jnp.maximum(h2 + b2_ref[...], 0.0)
    h3 = lax.dot_general(w3_ref[...], h2, dn,
                         preferred_element_type=jnp.float32)
    mlp = h3[0] + b3_ref[0]

    logits = lin + fm + const + mlp
    out_ref[...] = jax.nn.sigmoid(logits)


def _tc_call(xv2, xw2d, W1, b1c, W2, b2c, W3, b3, gam, bet, ew):
    vspec = pl.BlockSpec(memory_space=pltpu.VMEM)
    sspec = pl.BlockSpec(memory_space=pltpu.SMEM)
    return pl.pallas_call(
        _tc_body,
        out_shape=jax.ShapeDtypeStruct((B,), jnp.float32),
        in_specs=[vspec, vspec, vspec, vspec, vspec, vspec, vspec,
                  sspec, sspec, sspec, sspec],
        out_specs=vspec,
        scratch_shapes=[pltpu.VMEM((F * K, B), jnp.float32)],
    )(xv2, xw2d, W1, b1c, W2, b2c, W3, b3, gam, bet, ew)


def kernel(inputs, xw_table, xv_table, W1, b1, W2, b2, W3, b3,
           edge_weights, bn_gamma, bn_beta):
    idx_flat = inputs.reshape(B * F).astype(jnp.int32)
    xv_g, xw_g = _sc_gather(idx_flat, xv_table, xw_table)
    xv2 = xv_g.reshape(B, F * K)
    xw2d = xw_g.reshape(B, F)
    out = _tc_call(
        xv2, xw2d, W1, b1.reshape(-1, 1), W2, b2.reshape(-1, 1), W3,
        b3, bn_gamma, bn_beta, edge_weights)
    return out
